# baseline (speedup 1.0000x reference)
import functools

import jax
import jax.numpy as jnp
from jax import lax
from jax.experimental import pallas as pl
from jax.experimental.pallas import tpu as pltpu
from jax.experimental.pallas import tpu_sc as plsc


def _rne_hi16(x):
    b = lax.bitcast_convert_type(x, jnp.int32)
    return b + 0x7FFF + ((b >> 16) & 1)


def _coarse_body(li_ref, le_ref, w1_ref, wle_ref, out_ref):
    nb = le_ref.shape[1]
    oc = w1_ref.shape[1]
    ec = wle_ref.shape[1]
    z = jnp.dot(li_ref[...].astype(jnp.bfloat16), w1_ref[...],
                preferred_element_type=jnp.float32)
    les = [jnp.dot(le_ref[:, b, :].astype(jnp.bfloat16), wle_ref[...],
                   preferred_element_type=jnp.float32) for b in range(nb)]
    lo = jnp.concatenate([z, les[0]], axis=1)
    hi = jnp.concatenate(les[1:], axis=1)
    plo = _rne_hi16(lo)
    phi = _rne_hi16(hi)
    out_ref[...] = ((phi >> 16) << 16) | ((plo >> 16) & 0xFFFF)


def _fine_body(ce_ref, ci_ref, g_ref, wce_ref, w2_ref, w3_ref,
               b_ref, gam_ref, bet_ref, out_ref):
    f = ce_ref.shape[0]
    nb = ce_ref.shape[1]
    ec = wce_ref.shape[1]
    oc = w2_ref.shape[1]
    gw = g_ref[...]
    lo = lax.bitcast_convert_type(gw << 16, jnp.float32)
    hi = lax.bitcast_convert_type((gw >> 16) << 16, jnp.float32)
    gz = lo[:, :oc]
    gles = [lo[:, oc:oc + ec]] + [hi[:, b * ec:(b + 1) * ec] for b in range(nb - 1)]
    acc = gz + jnp.dot(ci_ref[...].astype(jnp.bfloat16), w2_ref[...],
                       preferred_element_type=jnp.float32)
    equ = None
    for b in range(nb):
        ceb = jnp.dot(ce_ref[:, b, :].astype(jnp.bfloat16), wce_ref[...],
                      preferred_element_type=jnp.float32)
        prod = ceb * gles[b]
        equ = prod if equ is None else equ + prod
    equ = equ * (1.0 / nb)
    acc = acc + jnp.dot(equ.astype(jnp.bfloat16), w3_ref[...],
                        preferred_element_type=jnp.float32)
    acc = acc + b_ref[...]
    groups = 32
    xg = acc.reshape(f, groups, oc // groups)
    mean = jnp.mean(xg, axis=-1, keepdims=True)
    var = jnp.mean(xg * xg, axis=-1, keepdims=True) - mean * mean
    xn = (xg - mean) * lax.rsqrt(var + 1e-5)
    x = xn.reshape(f, oc) * gam_ref[...] + bet_ref[...]
    out_ref[...] = jnp.where(x >= 0, x, 0.1 * x)


def _make_sc_gather(n_rows, d, dtype, chunk):
    info = plsc.get_sparse_core_info()
    nc, ns = info.num_cores, info.num_subcores
    nw = nc * ns
    b_per_w = n_rows // nw
    npair = b_per_w // (2 * chunk)
    mesh = plsc.VectorSubcoreMesh(core_axis_name="c", subcore_axis_name="s")

    @functools.partial(
        pl.kernel, mesh=mesh,
        out_type=jax.ShapeDtypeStruct((n_rows, d), dtype),
        scratch_types=[
            pltpu.VMEM((b_per_w,), jnp.int32),
            pltpu.VMEM((chunk, d), dtype),
            pltpu.VMEM((chunk, d), dtype),
            pltpu.SemaphoreType.DMA,
            pltpu.SemaphoreType.DMA,
        ],
    )
    def gk(table_hbm, idx_hbm, out_hbm, idx_v, rows0, rows1, sem0, sem1):
        wid = lax.axis_index("s") * nc + lax.axis_index("c")
        base = pl.multiple_of(wid * b_per_w, 8)
        pltpu.sync_copy(idx_hbm.at[pl.ds(base, b_per_w)], idx_v)

        def body(i, carry):
            o0 = pl.multiple_of(2 * i * chunk, 8)
            o1 = pl.multiple_of((2 * i + 1) * chunk, 8)
            g0 = pltpu.async_copy(
                table_hbm.at[idx_v.at[pl.ds(o0, chunk)]], rows0, sem0)
            g1 = pltpu.async_copy(
                table_hbm.at[idx_v.at[pl.ds(o1, chunk)]], rows1, sem1)
            g0.wait()
            pltpu.sync_copy(rows0, out_hbm.at[pl.ds(base + o0, chunk)])
            g1.wait()
            pltpu.sync_copy(rows1, out_hbm.at[pl.ds(base + o1, chunk)])
            return carry

        lax.fori_loop(0, npair, body, 0)

    return gk


def kernel(last_inv, cur_inv, last_equ, cur_equ, upsampling,
           W_last_equ, W_cur_equ, W_mlp, b_mlp, gamma, beta):
    n_c, inv_l = last_inv.shape
    n_f, inv_c = cur_inv.shape
    nb, equ_l = last_equ.shape[1], last_equ.shape[2]
    equ_c = cur_equ.shape[2]
    out_c = W_mlp.shape[1]
    dw = (out_c + nb * equ_c) // 2

    W1 = W_mlp[:inv_l].astype(jnp.bfloat16)
    W2 = W_mlp[inv_l:inv_l + inv_c].astype(jnp.bfloat16)
    W3 = W_mlp[inv_l + inv_c:].astype(jnp.bfloat16)
    Wle = W_last_equ.astype(jnp.bfloat16)
    Wce = W_cur_equ.astype(jnp.bfloat16)

    bc = 512
    table = pl.pallas_call(
        _coarse_body,
        grid=(pl.cdiv(n_c, bc),),
        in_specs=[
            pl.BlockSpec((bc, inv_l), lambda i: (i, 0)),
            pl.BlockSpec((bc, nb, equ_l), lambda i: (i, 0, 0)),
            pl.BlockSpec((inv_l, out_c), lambda i: (0, 0)),
            pl.BlockSpec((equ_l, equ_c), lambda i: (0, 0)),
        ],
        out_specs=pl.BlockSpec((bc, dw), lambda i: (i, 0)),
        out_shape=jax.ShapeDtypeStruct((n_c, dw), jnp.int32),
    )(last_inv, last_equ, W1, Wle)

    chunk = 56
    align = 32 * 2 * chunk
    n_pad = ((n_f + align - 1) // align) * align
    idx = upsampling[:, 0].astype(jnp.int32)
    idx = jnp.concatenate([idx, jnp.zeros((n_pad - n_f,), dtype=jnp.int32)])
    gathered = _make_sc_gather(n_pad, dw, jnp.int32, chunk)(table, idx)

    fb = 512
    out = pl.pallas_call(
        _fine_body,
        grid=(pl.cdiv(n_f, fb),),
        in_specs=[
            pl.BlockSpec((fb, nb, equ_c), lambda i: (i, 0, 0)),
            pl.BlockSpec((fb, inv_c), lambda i: (i, 0)),
            pl.BlockSpec((fb, dw), lambda i: (i, 0)),
            pl.BlockSpec((equ_c, equ_c), lambda i: (0, 0)),
            pl.BlockSpec((inv_c, out_c), lambda i: (0, 0)),
            pl.BlockSpec((equ_c, out_c), lambda i: (0, 0)),
            pl.BlockSpec((1, out_c), lambda i: (0, 0)),
            pl.BlockSpec((1, out_c), lambda i: (0, 0)),
            pl.BlockSpec((1, out_c), lambda i: (0, 0)),
        ],
        out_specs=pl.BlockSpec((fb, out_c), lambda i: (i, 0)),
        out_shape=jax.ShapeDtypeStruct((n_f, out_c), jnp.float32),
    )(cur_equ, cur_inv, gathered, Wce, W2, W3,
      b_mlp.reshape(1, out_c), gamma.reshape(1, out_c), beta.reshape(1, out_c))
    return out


# speedup vs baseline: 2.3049x; 2.3049x over previous
import functools

import jax
import jax.numpy as jnp
from jax import lax
from jax.experimental import pallas as pl
from jax.experimental.pallas import tpu as pltpu
from jax.experimental.pallas import tpu_sc as plsc


def _rne_hi16(x):
    b = lax.bitcast_convert_type(x, jnp.int32)
    return b + 0x7FFF + ((b >> 16) & 1)


def _coarse_body(li_ref, le_ref, w1_ref, wle_ref, out_ref):
    nb = le_ref.shape[1]
    oc = w1_ref.shape[1]
    ec = wle_ref.shape[1]
    z = jnp.dot(li_ref[...].astype(jnp.bfloat16), w1_ref[...],
                preferred_element_type=jnp.float32)
    les = [jnp.dot(le_ref[:, b, :].astype(jnp.bfloat16), wle_ref[...],
                   preferred_element_type=jnp.float32) for b in range(nb)]
    lo = jnp.concatenate([z, les[0]], axis=1)
    hi = jnp.concatenate(les[1:], axis=1)
    plo = _rne_hi16(lo)
    phi = _rne_hi16(hi)
    out_ref[...] = ((phi >> 16) << 16) | ((plo >> 16) & 0xFFFF)


def _fine_body(ce_ref, ci_ref, g_ref, wce_ref, w2_ref, w3_ref,
               b_ref, gam_ref, bet_ref, bmat_ref, out_ref):
    f = ce_ref.shape[0]
    nb = ce_ref.shape[1]
    ec = wce_ref.shape[1]
    oc = w2_ref.shape[1]
    gw = g_ref[...]
    lo = lax.bitcast_convert_type(gw << 16, jnp.float32)
    hi = lax.bitcast_convert_type((gw >> 16) << 16, jnp.float32)
    gz = lo[:, :oc]
    gles = [lo[:, oc:oc + ec]] + [hi[:, b * ec:(b + 1) * ec] for b in range(nb - 1)]
    acc = gz + jnp.dot(ci_ref[...].astype(jnp.bfloat16), w2_ref[...],
                       preferred_element_type=jnp.float32)
    equ = None
    for b in range(nb):
        ceb = jnp.dot(ce_ref[:, b, :].astype(jnp.bfloat16), wce_ref[...],
                      preferred_element_type=jnp.float32)
        prod = ceb * gles[b]
        equ = prod if equ is None else equ + prod
    equ = equ * (1.0 / nb)
    acc = acc + jnp.dot(equ.astype(jnp.bfloat16), w3_ref[...],
                        preferred_element_type=jnp.float32)
    acc = acc + b_ref[...]
    mu = jnp.dot(acc.astype(jnp.bfloat16), bmat_ref[...],
                 preferred_element_type=jnp.float32)
    e2 = jnp.dot((acc * acc).astype(jnp.bfloat16), bmat_ref[...],
                 preferred_element_type=jnp.float32)
    var = e2 - mu * mu
    x = (acc - mu) * lax.rsqrt(var + 1e-5) * gam_ref[...] + bet_ref[...]
    out_ref[...] = jnp.where(x >= 0, x, 0.1 * x)


def _make_sc_gather(n_rows, d, dtype, chunk):
    info = plsc.get_sparse_core_info()
    nc, ns = info.num_cores, info.num_subcores
    nw = nc * ns
    b_per_w = n_rows // nw
    npair = b_per_w // (2 * chunk)
    mesh = plsc.VectorSubcoreMesh(core_axis_name="c", subcore_axis_name="s")

    @functools.partial(
        pl.kernel, mesh=mesh,
        out_type=jax.ShapeDtypeStruct((n_rows, d), dtype),
        scratch_types=[
            pltpu.VMEM((b_per_w,), jnp.int32),
            pltpu.VMEM((chunk, d), dtype),
            pltpu.VMEM((chunk, d), dtype),
            pltpu.SemaphoreType.DMA,
            pltpu.SemaphoreType.DMA,
        ],
    )
    def gk(table_hbm, idx_hbm, out_hbm, idx_v, rows0, rows1, sem0, sem1):
        wid = lax.axis_index("s") * nc + lax.axis_index("c")
        base = pl.multiple_of(wid * b_per_w, 8)
        pltpu.sync_copy(idx_hbm.at[pl.ds(base, b_per_w)], idx_v)

        def body(i, carry):
            o0 = pl.multiple_of(2 * i * chunk, 8)
            o1 = pl.multiple_of((2 * i + 1) * chunk, 8)
            g0 = pltpu.async_copy(
                table_hbm.at[idx_v.at[pl.ds(o0, chunk)]], rows0, sem0)
            g1 = pltpu.async_copy(
                table_hbm.at[idx_v.at[pl.ds(o1, chunk)]], rows1, sem1)
            g0.wait()
            pltpu.sync_copy(rows0, out_hbm.at[pl.ds(base + o0, chunk)])
            g1.wait()
            pltpu.sync_copy(rows1, out_hbm.at[pl.ds(base + o1, chunk)])
            return carry

        lax.fori_loop(0, npair, body, 0)

    return gk


def kernel(last_inv, cur_inv, last_equ, cur_equ, upsampling,
           W_last_equ, W_cur_equ, W_mlp, b_mlp, gamma, beta):
    n_c, inv_l = last_inv.shape
    n_f, inv_c = cur_inv.shape
    nb, equ_l = last_equ.shape[1], last_equ.shape[2]
    equ_c = cur_equ.shape[2]
    out_c = W_mlp.shape[1]
    dw = (out_c + nb * equ_c) // 2

    W1 = W_mlp[:inv_l].astype(jnp.bfloat16)
    W2 = W_mlp[inv_l:inv_l + inv_c].astype(jnp.bfloat16)
    W3 = W_mlp[inv_l + inv_c:].astype(jnp.bfloat16)
    Wle = W_last_equ.astype(jnp.bfloat16)
    Wce = W_cur_equ.astype(jnp.bfloat16)
    grp = jnp.arange(out_c, dtype=jnp.int32) // 16
    bmat = jnp.where(grp[:, None] == grp[None, :], 1.0 / 16, 0.0
                     ).astype(jnp.bfloat16)

    bc = 512
    table = pl.pallas_call(
        _coarse_body,
        grid=(pl.cdiv(n_c, bc),),
        in_specs=[
            pl.BlockSpec((bc, inv_l), lambda i: (i, 0)),
            pl.BlockSpec((bc, nb, equ_l), lambda i: (i, 0, 0)),
            pl.BlockSpec((inv_l, out_c), lambda i: (0, 0)),
            pl.BlockSpec((equ_l, equ_c), lambda i: (0, 0)),
        ],
        out_specs=pl.BlockSpec((bc, dw), lambda i: (i, 0)),
        out_shape=jax.ShapeDtypeStruct((n_c, dw), jnp.int32),
    )(last_inv, last_equ, W1, Wle)

    chunk = 56
    align = 32 * 2 * chunk
    n_pad = ((n_f + align - 1) // align) * align
    idx = upsampling[:, 0].astype(jnp.int32)
    idx = jnp.concatenate([idx, jnp.zeros((n_pad - n_f,), dtype=jnp.int32)])
    gathered = _make_sc_gather(n_pad, dw, jnp.int32, chunk)(table, idx)

    fb = 512
    out = pl.pallas_call(
        _fine_body,
        grid=(pl.cdiv(n_f, fb),),
        in_specs=[
            pl.BlockSpec((fb, nb, equ_c), lambda i: (i, 0, 0)),
            pl.BlockSpec((fb, inv_c), lambda i: (i, 0)),
            pl.BlockSpec((fb, dw), lambda i: (i, 0)),
            pl.BlockSpec((equ_c, equ_c), lambda i: (0, 0)),
            pl.BlockSpec((inv_c, out_c), lambda i: (0, 0)),
            pl.BlockSpec((equ_c, out_c), lambda i: (0, 0)),
            pl.BlockSpec((1, out_c), lambda i: (0, 0)),
            pl.BlockSpec((1, out_c), lambda i: (0, 0)),
            pl.BlockSpec((1, out_c), lambda i: (0, 0)),
            pl.BlockSpec((out_c, out_c), lambda i: (0, 0)),
        ],
        out_specs=pl.BlockSpec((fb, out_c), lambda i: (i, 0)),
        out_shape=jax.ShapeDtypeStruct((n_f, out_c), jnp.float32),
    )(cur_equ, cur_inv, gathered, Wce, W2, W3,
      b_mlp.reshape(1, out_c), gamma.reshape(1, out_c), beta.reshape(1, out_c),
      bmat)
    return out


# speedup vs baseline: 2.7503x; 1.1932x over previous
import functools

import jax
import jax.numpy as jnp
from jax import lax
from jax.experimental import pallas as pl
from jax.experimental.pallas import tpu as pltpu
from jax.experimental.pallas import tpu_sc as plsc


def _rne_hi16(x):
    b = lax.bitcast_convert_type(x, jnp.int32)
    return b + 0x7FFF + ((b >> 16) & 1)


def _coarse_body(li_ref, le_ref, w1_ref, wle_ref, out_ref):
    nb = le_ref.shape[1]
    z = jnp.dot(li_ref[...].astype(jnp.bfloat16), w1_ref[...],
                preferred_element_type=jnp.float32)
    le_all = le_ref[...].astype(jnp.bfloat16)
    les = [jnp.dot(le_all[:, b, :], wle_ref[...],
                   preferred_element_type=jnp.float32) for b in range(nb)]
    lo = jnp.concatenate([z, les[0]], axis=1)
    hi = jnp.concatenate(les[1:], axis=1)
    plo = _rne_hi16(lo)
    phi = _rne_hi16(hi)
    out_ref[...] = ((phi >> 16) << 16) | ((plo >> 16) & 0xFFFF)


def _fine_body(ce_ref, ci_ref, g_ref, wce_ref, w2_ref, w3_ref,
               b_ref, gam_ref, bet_ref, bmat_ref, out_ref):
    nb = ce_ref.shape[1]
    ec = wce_ref.shape[1]
    oc = w2_ref.shape[1]
    ce_all = ce_ref[...].astype(jnp.bfloat16)
    gw = g_ref[...]
    lo = lax.bitcast_convert_type(gw << 16, jnp.float32)
    hi = lax.bitcast_convert_type((gw >> 16) << 16, jnp.float32)
    gz = lo[:, :oc]
    gles = [lo[:, oc:oc + ec]] + [hi[:, b * ec:(b + 1) * ec] for b in range(nb - 1)]
    acc = gz + jnp.dot(ci_ref[...].astype(jnp.bfloat16), w2_ref[...],
                       preferred_element_type=jnp.float32)
    equ = None
    for b in range(nb):
        ceb = jnp.dot(ce_all[:, b, :], wce_ref[...],
                      preferred_element_type=jnp.float32)
        prod = ceb * gles[b]
        equ = prod if equ is None else equ + prod
    equ = equ * (1.0 / nb)
    acc = acc + jnp.dot(equ.astype(jnp.bfloat16), w3_ref[...],
                        preferred_element_type=jnp.float32)
    acc = acc + b_ref[...]
    mu = jnp.dot(acc.astype(jnp.bfloat16), bmat_ref[...],
                 preferred_element_type=jnp.float32)
    e2 = jnp.dot((acc * acc).astype(jnp.bfloat16), bmat_ref[...],
                 preferred_element_type=jnp.float32)
    var = e2 - mu * mu
    x = (acc - mu) * lax.rsqrt(var + 1e-5) * gam_ref[...] + bet_ref[...]
    out_ref[...] = jnp.where(x >= 0, x, 0.1 * x)


def _make_sc_gather(n_rows, d, dtype, chunk):
    info = plsc.get_sparse_core_info()
    nc, ns = info.num_cores, info.num_subcores
    nw = nc * ns
    b_per_w = n_rows // nw
    npair = b_per_w // (2 * chunk)
    mesh = plsc.VectorSubcoreMesh(core_axis_name="c", subcore_axis_name="s")

    @functools.partial(
        pl.kernel, mesh=mesh,
        out_type=jax.ShapeDtypeStruct((n_rows, d), dtype),
        scratch_types=[
            pltpu.VMEM((b_per_w,), jnp.int32),
            pltpu.VMEM((chunk, d), dtype),
            pltpu.VMEM((chunk, d), dtype),
            pltpu.SemaphoreType.DMA,
            pltpu.SemaphoreType.DMA,
        ],
    )
    def gk(table_hbm, idx_hbm, out_hbm, idx_v, rows0, rows1, sem0, sem1):
        wid = lax.axis_index("s") * nc + lax.axis_index("c")
        base = pl.multiple_of(wid * b_per_w, 8)
        pltpu.sync_copy(idx_hbm.at[pl.ds(base, b_per_w)], idx_v)

        def body(i, carry):
            o0 = pl.multiple_of(2 * i * chunk, 8)
            o1 = pl.multiple_of((2 * i + 1) * chunk, 8)
            g0 = pltpu.async_copy(
                table_hbm.at[idx_v.at[pl.ds(o0, chunk)]], rows0, sem0)
            g1 = pltpu.async_copy(
                table_hbm.at[idx_v.at[pl.ds(o1, chunk)]], rows1, sem1)
            g0.wait()
            pltpu.sync_copy(rows0, out_hbm.at[pl.ds(base + o0, chunk)])
            g1.wait()
            pltpu.sync_copy(rows1, out_hbm.at[pl.ds(base + o1, chunk)])
            return carry

        lax.fori_loop(0, npair, body, 0)

    return gk


def kernel(last_inv, cur_inv, last_equ, cur_equ, upsampling,
           W_last_equ, W_cur_equ, W_mlp, b_mlp, gamma, beta):
    n_c, inv_l = last_inv.shape
    n_f, inv_c = cur_inv.shape
    nb, equ_l = last_equ.shape[1], last_equ.shape[2]
    equ_c = cur_equ.shape[2]
    out_c = W_mlp.shape[1]
    dw = (out_c + nb * equ_c) // 2

    W1 = W_mlp[:inv_l].astype(jnp.bfloat16)
    W2 = W_mlp[inv_l:inv_l + inv_c].astype(jnp.bfloat16)
    W3 = W_mlp[inv_l + inv_c:].astype(jnp.bfloat16)
    Wle = W_last_equ.astype(jnp.bfloat16)
    Wce = W_cur_equ.astype(jnp.bfloat16)
    grp = jnp.arange(out_c, dtype=jnp.int32) // 16
    bmat = jnp.where(grp[:, None] == grp[None, :], 1.0 / 16, 0.0
                     ).astype(jnp.bfloat16)

    bc = 512
    table = pl.pallas_call(
        _coarse_body,
        grid=(pl.cdiv(n_c, bc),),
        in_specs=[
            pl.BlockSpec((bc, inv_l), lambda i: (i, 0)),
            pl.BlockSpec((bc, nb, equ_l), lambda i: (i, 0, 0)),
            pl.BlockSpec((inv_l, out_c), lambda i: (0, 0)),
            pl.BlockSpec((equ_l, equ_c), lambda i: (0, 0)),
        ],
        out_specs=pl.BlockSpec((bc, dw), lambda i: (i, 0)),
        out_shape=jax.ShapeDtypeStruct((n_c, dw), jnp.int32),
    )(last_inv, last_equ, W1, Wle)

    chunk = 56
    align = 32 * 2 * chunk
    n_pad = ((n_f + align - 1) // align) * align
    idx = upsampling[:, 0].astype(jnp.int32)
    idx = jnp.concatenate([idx, jnp.zeros((n_pad - n_f,), dtype=jnp.int32)])
    gathered = _make_sc_gather(n_pad, dw, jnp.int32, chunk)(table, idx)

    fb = 1024
    out = pl.pallas_call(
        _fine_body,
        grid=(pl.cdiv(n_f, fb),),
        in_specs=[
            pl.BlockSpec((fb, nb, equ_c), lambda i: (i, 0, 0)),
            pl.BlockSpec((fb, inv_c), lambda i: (i, 0)),
            pl.BlockSpec((fb, dw), lambda i: (i, 0)),
            pl.BlockSpec((equ_c, equ_c), lambda i: (0, 0)),
            pl.BlockSpec((inv_c, out_c), lambda i: (0, 0)),
            pl.BlockSpec((equ_c, out_c), lambda i: (0, 0)),
            pl.BlockSpec((1, out_c), lambda i: (0, 0)),
            pl.BlockSpec((1, out_c), lambda i: (0, 0)),
            pl.BlockSpec((1, out_c), lambda i: (0, 0)),
            pl.BlockSpec((out_c, out_c), lambda i: (0, 0)),
        ],
        out_specs=pl.BlockSpec((fb, out_c), lambda i: (i, 0)),
        out_shape=jax.ShapeDtypeStruct((n_f, out_c), jnp.float32),
    )(cur_equ, cur_inv, gathered, Wce, W2, W3,
      b_mlp.reshape(1, out_c), gamma.reshape(1, out_c), beta.reshape(1, out_c),
      bmat)
    return out
